# baseline (speedup 1.0000x reference)
import functools

import jax
import jax.numpy as jnp
from jax import lax
from jax.experimental import pallas as pl
from jax.experimental.pallas import tpu as pltpu
from jax.experimental.pallas import tpu_sc as plsc

_NC = 2
_NS = 16
_NW = _NC * _NS


@functools.lru_cache(maxsize=None)
def _make_gather(B: int, D: int, DP: int, C: int, NBUF: int, SL: int):
    bpw = B // _NW
    nchunk = bpw // C
    ngrp = nchunk // NBUF
    assert B % _NW == 0 and bpw % C == 0 and nchunk % NBUF == 0
    assert ngrp >= 2 and 1 <= SL < NBUF and C % 8 == 0
    mesh = plsc.VectorSubcoreMesh(core_axis_name="c", subcore_axis_name="s")

    @functools.partial(
        pl.kernel,
        mesh=mesh,
        out_type=jax.ShapeDtypeStruct((B, D), jnp.float32),
        scratch_types=[
            pltpu.VMEM((bpw,), jnp.int32),
            pltpu.VMEM((NBUF, C, DP), jnp.float32),
            pltpu.VMEM((NBUF, C, D), jnp.float32),
        ]
        + [pltpu.SemaphoreType.DMA] * (2 * NBUF),
    )
    def emb_gather(idx_hbm, table_hbm, out_hbm, idx_v, rows_g, rows_s, *sems):
        gsem = sems[:NBUF]
        ssem = sems[NBUF:]
        wid = lax.axis_index("s") * _NC + lax.axis_index("c")
        base = wid * bpw

        pltpu.sync_copy(idx_hbm.at[pl.ds(base, bpw)], idx_v)

        def start_gather(c, b):
            pltpu.async_copy(
                table_hbm.at[idx_v.at[pl.ds(c * C, C)]], rows_g.at[b], gsem[b]
            )

        def repack(b):
            def rows8(i, carry):
                r0 = i * 8
                for dr in range(8):
                    for k in range(D // 16):
                        rows_s[b, r0 + dr, pl.ds(k * 16, 16)] = (
                            rows_g[b, r0 + dr, pl.ds(k * 16, 16)]
                        )
                return carry
            lax.fori_loop(0, C // 8, rows8, 0)

        def start_store(c, b):
            pltpu.async_copy(
                rows_s.at[b], out_hbm.at[pl.ds(base + c * C, C)], ssem[b]
            )

        def wait_g(b):
            pltpu.make_async_copy(rows_g.at[b], out_hbm.at[pl.ds(0, C)],
                                  gsem[b]).wait()

        def wait_s(b):
            pltpu.make_async_copy(rows_s.at[b], out_hbm.at[pl.ds(0, C)],
                                  ssem[b]).wait()

        for b in range(NBUF):
            start_gather(b, b)
            if b >= SL:
                wait_g(b - SL)
                repack(b - SL)
                start_store(b - SL, b - SL)

        def group(g, carry):
            c0 = g * NBUF
            for b in range(NBUF):
                wait_s(b)
                start_gather(c0 + b, b)
                b2 = (b - SL) % NBUF
                wait_g(b2)
                repack(b2)
                start_store(c0 + b - SL, b2)
            return carry

        lax.fori_loop(1, ngrp, group, 0)

        for k in range(SL):
            c = nchunk - SL + k
            b = c % NBUF
            wait_g(b)
            repack(b)
            start_store(c, b)
        for b in range(NBUF):
            wait_s(b)

    return emb_gather


def kernel(x, emb_loc):
    R, T = x.shape
    V, D = emb_loc.shape
    DP = 128
    VP = (V + 7) // 8 * 8
    emb_p = jnp.pad(emb_loc, ((0, VP - V), (0, DP - D)))
    xf = x.reshape(-1).astype(jnp.int32)
    out = _make_gather(R * T, D, DP, 80, 5, 2)(xf, emb_p)
    return out.reshape(R, T, D)
